# baseline (speedup 1.0000x reference)
import functools

import jax
import jax.numpy as jnp
import numpy as np
from jax import lax
from jax.experimental import pallas as pl
from jax.experimental.pallas import tpu as pltpu
from jax.experimental.pallas import tpu_sc as plsc

NC = 2
NS = 16
NW = NC * NS

B = 16384
S = 20
D = 64
VT = 2001
VTITLE = 1001
RPW = B // NW
CB = 32
NCHUNK = RPW // CB
PW = D // 2
TPW = S // 2
OTILE = CB * 2 * D
MASK_HI = np.int32(-65536)


def _treemax(vals):
    while len(vals) > 1:
        nxt = [jnp.maximum(a, b) for a, b in zip(vals[::2], vals[1::2])]
        if len(vals) % 2:
            nxt.append(vals[-1])
        vals = nxt
    return vals[0]


def _unpack_store(ai, ref, base):
    ref[pl.ds(base, 16)] = plsc.bitcast(jnp.left_shift(ai, 16), jnp.float32)
    ref[pl.ds(base + 16, 16)] = plsc.bitcast(
        jnp.bitwise_and(ai, MASK_HI), jnp.float32)


def _sc_body(title_ids_hbm, tok_hbm, title_tab_hbm, text_tab_hbm,
             out_hbm, textv, titlev, pkv, tidv, outv, sem, sem_tab):
    wid = lax.axis_index("s") * NC + lax.axis_index("c")

    pltpu.sync_copy(tok_hbm.at[pl.ds(wid * (RPW * TPW), RPW * TPW)], pkv)
    pltpu.sync_copy(title_ids_hbm.at[pl.ds(wid * RPW, RPW)], tidv)
    text_dma = pltpu.async_copy(text_tab_hbm, textv, sem_tab)
    title_dma = pltpu.async_copy(title_tab_hbm, titlev, sem_tab)

    def remap_pack(i, _):
        w = pkv[pl.ds(i * 16, 16)]
        e = jnp.bitwise_and(w, jnp.int32(0xFFFF))
        o = lax.shift_right_logical(w, 16)
        e = jnp.where(e == 0, jnp.int32(VT - 1), e) * PW
        o = jnp.where(o == 0, jnp.int32(VT - 1), o) * PW
        pkv[pl.ds(i * 16, 16)] = jnp.bitwise_or(e, jnp.left_shift(o, 16))
        return _

    lax.fori_loop(0, RPW * TPW // 16, remap_pack, None)

    def scale_tid(i, _):
        tidv[pl.ds(i * 16, 16)] = tidv[pl.ds(i * 16, 16)] * PW
        return _

    lax.fori_loop(0, RPW // 16, scale_tid, None)

    text_dma.wait()
    title_dma.wait()

    def chunk_body(g, _):
        ob0 = lax.rem(g, 2) * OTILE

        @pl.when(g >= 2)
        def _wait_prev():
            pltpu.make_async_copy(
                outv.at[pl.ds(0, OTILE)], out_hbm.at[pl.ds(0, OTILE)],
                sem).wait()

        def grp_body(gg, __):
            row0 = g * CB + gg * 16
            tvec = tidv[pl.ds(row0, 16)]
            pv = [pkv[pl.ds(row0 * TPW + j * 16, 16)] for j in range(TPW)]
            for i in range(16):
                offs = []
                for k in range(TPW):
                    p = i * TPW + k
                    w = pv[p // 16][p % 16]
                    offs.append(jnp.bitwise_and(w, jnp.int32(0xFFFF)))
                    offs.append(lax.shift_right_logical(w, 16))
                ob = ob0 + (gg * 16 + i) * 2 * D
                ta = tvec[i]
                for h in (0, 1):
                    vals = [
                        plsc.bitcast(textv[pl.ds(offs[s] + h * 16, 16)],
                                     jnp.bfloat16)
                        for s in range(S)
                    ]
                    acc = plsc.bitcast(_treemax(vals), jnp.int32)
                    _unpack_store(acc, outv, ob + D + h * 32)
                    _unpack_store(titlev[pl.ds(ta + h * 16, 16)], outv,
                                  ob + h * 32)
            return __

        lax.fori_loop(0, CB // 16, grp_body, None)
        pltpu.async_copy(
            outv.at[pl.ds(ob0, OTILE)],
            out_hbm.at[pl.ds((wid * RPW + g * CB) * 2 * D, OTILE)], sem)
        return _

    lax.fori_loop(0, NCHUNK, chunk_body, None)
    for _ in range(2):
        pltpu.make_async_copy(
            outv.at[pl.ds(0, OTILE)], out_hbm.at[pl.ds(0, OTILE)], sem).wait()


def _permute(table_f32):
    v = table_f32.shape[0]
    t = table_f32.astype(jnp.bfloat16).reshape(v, 2, 2, 16)
    perm = t.transpose(0, 1, 3, 2).reshape(v, PW, 2)
    return lax.bitcast_convert_type(perm, jnp.int32).reshape(-1)


@jax.jit
def _run(title_ids, tok_pk, title_perm, text_perm):
    mesh = plsc.VectorSubcoreMesh(core_axis_name="c", subcore_axis_name="s")
    f = functools.partial(
        pl.kernel,
        out_type=jax.ShapeDtypeStruct((B * 2 * D,), jnp.float32),
        mesh=mesh,
        compiler_params=pltpu.CompilerParams(
            needs_layout_passes=False, disable_bounds_checks=True),
        scratch_types=[
            pltpu.VMEM((VT * PW,), jnp.int32),
            pltpu.VMEM((VTITLE * PW,), jnp.int32),
            pltpu.VMEM((RPW * TPW,), jnp.int32),
            pltpu.VMEM((RPW,), jnp.int32),
            pltpu.VMEM((2 * OTILE,), jnp.float32),
            pltpu.SemaphoreType.DMA,
            pltpu.SemaphoreType.DMA,
        ],
    )(_sc_body)
    return f(title_ids, tok_pk, title_perm, text_perm)


def kernel(title_ids, token_ids, title_table, text_table):
    text_aug = jnp.concatenate(
        [text_table, jnp.full((1, D), -1e9, jnp.float32)], axis=0)
    text_perm = _permute(text_aug)
    title_perm = _permute(title_table)
    tok_pk = lax.bitcast_convert_type(
        token_ids.astype(jnp.int16).reshape(-1, 2), jnp.int32)
    out = _run(title_ids, tok_pk, title_perm, text_perm)
    return out.reshape(B, 2 * D)


# speedup vs baseline: 1.8605x; 1.8605x over previous
import functools

import jax
import jax.numpy as jnp
import numpy as np
from jax import lax
from jax.experimental import pallas as pl
from jax.experimental.pallas import tpu as pltpu
from jax.experimental.pallas import tpu_sc as plsc

NC = 2
NS = 16
NW = NC * NS

B = 16384
S = 20
D = 64
VT = 2001
VTITLE = 1001
RPW = B // NW
CB = 32
NCHUNK = RPW // CB
PW = D // 2
TPW = S // 2
OTILE = CB * 2 * D
MASK_HI = np.int32(-65536)


def _treemax(vals):
    while len(vals) > 1:
        nxt = [jnp.maximum(a, b) for a, b in zip(vals[::2], vals[1::2])]
        if len(vals) % 2:
            nxt.append(vals[-1])
        vals = nxt
    return vals[0]


def _unpack_store(ai, ref, base):
    ref[pl.ds(base, 16)] = plsc.bitcast(jnp.left_shift(ai, 16), jnp.float32)
    ref[pl.ds(base + 16, 16)] = plsc.bitcast(
        jnp.bitwise_and(ai, MASK_HI), jnp.float32)


def _sc_body(title_ids_hbm, tok_hbm, title_tab_hbm, text_tab_hbm,
             out_hbm, textv, titlev, rawv, pkv, tidv, outv, sem, sem_tab):
    wid = lax.axis_index("s") * NC + lax.axis_index("c")

    pltpu.sync_copy(tok_hbm.at[pl.ds(wid * (RPW * S), RPW * S)], rawv)
    pltpu.sync_copy(title_ids_hbm.at[pl.ds(wid * RPW, RPW)], tidv)
    text_dma = pltpu.async_copy(text_tab_hbm, textv, sem_tab)
    title_dma = pltpu.async_copy(title_tab_hbm, titlev, sem_tab)

    iota = lax.iota(jnp.int32, 16)
    idxe = jnp.bitwise_and(iota * 2, jnp.int32(15))
    idxo = jnp.bitwise_and(iota * 2 + 1, jnp.int32(15))
    lane_lo = iota < 8
    _dn = lax.GatherDimensionNumbers(
        offset_dims=(), collapsed_slice_dims=(0,), start_index_map=(0,))

    def _lanegather(v, i):
        return lax.gather(v, i[:, None], _dn, slice_sizes=(1,),
                          mode=lax.GatherScatterMode.PROMISE_IN_BOUNDS)

    def remap_pack(i, _):
        va = rawv[pl.ds(i * 32, 16)]
        vb = rawv[pl.ds(i * 32 + 16, 16)]
        e = jnp.where(lane_lo, _lanegather(va, idxe), _lanegather(vb, idxe))
        o = jnp.where(lane_lo, _lanegather(va, idxo), _lanegather(vb, idxo))
        e = jnp.where(e == 0, jnp.int32(VT - 1), e) * PW
        o = jnp.where(o == 0, jnp.int32(VT - 1), o) * PW
        pkv[pl.ds(i * 16, 16)] = jnp.bitwise_or(e, jnp.left_shift(o, 16))
        return _

    lax.fori_loop(0, RPW * TPW // 16, remap_pack, None)

    def scale_tid(i, _):
        tidv[pl.ds(i * 16, 16)] = tidv[pl.ds(i * 16, 16)] * PW
        return _

    lax.fori_loop(0, RPW // 16, scale_tid, None)

    text_dma.wait()
    title_dma.wait()

    def chunk_body(g, _):
        ob0 = lax.rem(g, 2) * OTILE

        @pl.when(g >= 2)
        def _wait_prev():
            pltpu.make_async_copy(
                outv.at[pl.ds(0, OTILE)], out_hbm.at[pl.ds(0, OTILE)],
                sem).wait()

        def grp_body(gg, __):
            row0 = g * CB + gg * 16
            tvec = tidv[pl.ds(row0, 16)]
            pv = [pkv[pl.ds(row0 * TPW + j * 16, 16)] for j in range(TPW)]
            for i in range(16):
                offs = []
                for k in range(TPW):
                    p = i * TPW + k
                    w = pv[p // 16][p % 16]
                    offs.append(jnp.bitwise_and(w, jnp.int32(0xFFFF)))
                    offs.append(lax.shift_right_logical(w, 16))
                ob = ob0 + (gg * 16 + i) * 2 * D
                ta = tvec[i]
                for h in (0, 1):
                    vals = [
                        plsc.bitcast(textv[pl.ds(offs[s] + h * 16, 16)],
                                     jnp.bfloat16)
                        for s in range(S)
                    ]
                    acc = plsc.bitcast(_treemax(vals), jnp.int32)
                    _unpack_store(acc, outv, ob + D + h * 32)
                    _unpack_store(titlev[pl.ds(ta + h * 16, 16)], outv,
                                  ob + h * 32)
            return __

        lax.fori_loop(0, CB // 16, grp_body, None)
        pltpu.async_copy(
            outv.at[pl.ds(ob0, OTILE)],
            out_hbm.at[pl.ds((wid * RPW + g * CB) * 2 * D, OTILE)], sem)
        return _

    lax.fori_loop(0, NCHUNK, chunk_body, None)
    for _ in range(2):
        pltpu.make_async_copy(
            outv.at[pl.ds(0, OTILE)], out_hbm.at[pl.ds(0, OTILE)], sem).wait()


def _permute(table_f32):
    v = table_f32.shape[0]
    t = table_f32.astype(jnp.bfloat16).reshape(v, 2, 2, 16)
    perm = t.transpose(0, 1, 3, 2).reshape(v, PW, 2)
    return lax.bitcast_convert_type(perm, jnp.int32).reshape(-1)


@jax.jit
def _run(title_ids, tok_pk, title_perm, text_perm):
    mesh = plsc.VectorSubcoreMesh(core_axis_name="c", subcore_axis_name="s")
    f = functools.partial(
        pl.kernel,
        out_type=jax.ShapeDtypeStruct((B * 2 * D,), jnp.float32),
        mesh=mesh,
        compiler_params=pltpu.CompilerParams(
            needs_layout_passes=False, disable_bounds_checks=True),
        scratch_types=[
            pltpu.VMEM((VT * PW,), jnp.int32),
            pltpu.VMEM((VTITLE * PW,), jnp.int32),
            pltpu.VMEM((RPW * S,), jnp.int32),
            pltpu.VMEM((RPW * TPW,), jnp.int32),
            pltpu.VMEM((RPW,), jnp.int32),
            pltpu.VMEM((2 * OTILE,), jnp.float32),
            pltpu.SemaphoreType.DMA,
            pltpu.SemaphoreType.DMA,
        ],
    )(_sc_body)
    return f(title_ids, tok_pk, title_perm, text_perm)


def kernel(title_ids, token_ids, title_table, text_table):
    text_aug = jnp.concatenate(
        [text_table, jnp.full((1, D), -1e9, jnp.float32)], axis=0)
    text_perm = _permute(text_aug)
    title_perm = _permute(title_table)
    out = _run(title_ids, token_ids.reshape(-1), title_perm, text_perm)
    return out.reshape(B, 2 * D)


# speedup vs baseline: 1.9499x; 1.0481x over previous
import functools

import jax
import jax.numpy as jnp
import numpy as np
from jax import lax
from jax.experimental import pallas as pl
from jax.experimental.pallas import tpu as pltpu
from jax.experimental.pallas import tpu_sc as plsc

NC = 2
NS = 16
NW = NC * NS

B = 16384
S = 20
D = 64
VT = 2001
VTITLE = 1001
RPW = B // NW
CB = 32
NCHUNK = RPW // CB
PW = D // 2
TPW = S // 2
OTILE = CB * 2 * D
MASK_HI = np.int32(-65536)


def _treemax(vals):
    while len(vals) > 1:
        nxt = [jnp.maximum(a, b) for a, b in zip(vals[::2], vals[1::2])]
        if len(vals) % 2:
            nxt.append(vals[-1])
        vals = nxt
    return vals[0]


def _unpack_store(ai, ref, idx2, base):
    plsc.store_scatter(ref, [idx2 + base],
                       plsc.bitcast(jnp.left_shift(ai, 16), jnp.float32))
    plsc.store_scatter(ref, [idx2 + (base + 1)],
                       plsc.bitcast(jnp.bitwise_and(ai, MASK_HI), jnp.float32))


def _sc_body(title_ids_hbm, tok_hbm, title_tab_hbm, text_tab_hbm,
             out_hbm, textv, titlev, rawv, pkv, tidv, outv, sem, sem_tab):
    wid = lax.axis_index("s") * NC + lax.axis_index("c")

    pltpu.sync_copy(tok_hbm.at[pl.ds(wid * (RPW * S), RPW * S)], rawv)
    pltpu.sync_copy(title_ids_hbm.at[pl.ds(wid * RPW, RPW)], tidv)
    text_dma = pltpu.async_copy(text_tab_hbm, textv, sem_tab)
    title_dma = pltpu.async_copy(title_tab_hbm, titlev, sem_tab)

    iota = lax.iota(jnp.int32, 16)
    idxae = jnp.bitwise_and(iota * 2, jnp.int32(15))
    idxao = jnp.bitwise_and(iota * 2 + 1, jnp.int32(15))
    lane_lo = iota < 8
    _dn = lax.GatherDimensionNumbers(
        offset_dims=(), collapsed_slice_dims=(0,), start_index_map=(0,))

    def _lanegather(v, i):
        return lax.gather(v, i[:, None], _dn, slice_sizes=(1,),
                          mode=lax.GatherScatterMode.PROMISE_IN_BOUNDS)

    def remap_pack(i, _):
        va = rawv[pl.ds(i * 32, 16)]
        vb = rawv[pl.ds(i * 32 + 16, 16)]
        e = jnp.where(lane_lo, _lanegather(va, idxae), _lanegather(vb, idxae))
        o = jnp.where(lane_lo, _lanegather(va, idxao), _lanegather(vb, idxao))
        e = jnp.where(e == 0, jnp.int32(VT - 1), e) * PW
        o = jnp.where(o == 0, jnp.int32(VT - 1), o) * PW
        pkv[pl.ds(i * 16, 16)] = jnp.bitwise_or(e, jnp.left_shift(o, 16))
        return _

    lax.fori_loop(0, RPW * TPW // 16, remap_pack, None)

    def scale_tid(i, _):
        tidv[pl.ds(i * 16, 16)] = tidv[pl.ds(i * 16, 16)] * PW
        return _

    lax.fori_loop(0, RPW // 16, scale_tid, None)

    text_dma.wait()
    title_dma.wait()

    def chunk_body(g, _):
        ob0 = lax.rem(g, 2) * OTILE

        @pl.when(g >= 2)
        def _wait_prev():
            pltpu.make_async_copy(
                outv.at[pl.ds(0, OTILE)], out_hbm.at[pl.ds(0, OTILE)],
                sem).wait()

        def grp_body(gg, __):
            row0 = g * CB + gg * 16
            tvec = tidv[pl.ds(row0, 16)]
            idx2 = iota * 2
            pv = [pkv[pl.ds(row0 * TPW + j * 16, 16)] for j in range(TPW)]
            for i in range(16):
                offs = []
                for k in range(TPW):
                    p = i * TPW + k
                    w = pv[p // 16][p % 16]
                    offs.append(jnp.bitwise_and(w, jnp.int32(0xFFFF)))
                    offs.append(lax.shift_right_logical(w, 16))
                ob = ob0 + (gg * 16 + i) * 2 * D
                ta = tvec[i]
                for h in (0, 1):
                    vals = [
                        plsc.bitcast(textv[pl.ds(offs[s] + h * 16, 16)],
                                     jnp.bfloat16)
                        for s in range(S)
                    ]
                    acc = plsc.bitcast(_treemax(vals), jnp.int32)
                    _unpack_store(acc, outv, idx2, ob + D + h * 32)
                    _unpack_store(titlev[pl.ds(ta + h * 16, 16)], outv,
                                  idx2, ob + h * 32)
            return __

        lax.fori_loop(0, CB // 16, grp_body, None)
        pltpu.async_copy(
            outv.at[pl.ds(ob0, OTILE)],
            out_hbm.at[pl.ds((wid * RPW + g * CB) * 2 * D, OTILE)], sem)
        return _

    lax.fori_loop(0, NCHUNK, chunk_body, None)
    for _ in range(2):
        pltpu.make_async_copy(
            outv.at[pl.ds(0, OTILE)], out_hbm.at[pl.ds(0, OTILE)], sem).wait()


def _packtab(table_f32):
    v = table_f32.shape[0]
    t = table_f32.astype(jnp.bfloat16).reshape(v, PW, 2)
    return lax.bitcast_convert_type(t, jnp.int32).reshape(-1)


@jax.jit
def _run(title_ids, tok_pk, title_perm, text_perm):
    mesh = plsc.VectorSubcoreMesh(core_axis_name="c", subcore_axis_name="s")
    f = functools.partial(
        pl.kernel,
        out_type=jax.ShapeDtypeStruct((B * 2 * D,), jnp.float32),
        mesh=mesh,
        compiler_params=pltpu.CompilerParams(
            needs_layout_passes=False, disable_bounds_checks=True),
        scratch_types=[
            pltpu.VMEM((VT * PW,), jnp.int32),
            pltpu.VMEM((VTITLE * PW,), jnp.int32),
            pltpu.VMEM((RPW * S,), jnp.int32),
            pltpu.VMEM((RPW * TPW,), jnp.int32),
            pltpu.VMEM((RPW,), jnp.int32),
            pltpu.VMEM((2 * OTILE,), jnp.float32),
            pltpu.SemaphoreType.DMA,
            pltpu.SemaphoreType.DMA,
        ],
    )(_sc_body)
    return f(title_ids, tok_pk, title_perm, text_perm)


def kernel(title_ids, token_ids, title_table, text_table):
    text_aug = jnp.concatenate(
        [text_table, jnp.full((1, D), -1e9, jnp.float32)], axis=0)
    text_perm = _packtab(text_aug)
    title_perm = _packtab(title_table)
    out = _run(title_ids, token_ids.reshape(-1), title_perm, text_perm)
    return out.reshape(B, 2 * D)


# speedup vs baseline: 2.2362x; 1.1468x over previous
import functools

import jax
import jax.numpy as jnp
import numpy as np
from jax import lax
from jax.experimental import pallas as pl
from jax.experimental.pallas import tpu as pltpu
from jax.experimental.pallas import tpu_sc as plsc

NC = 2
NS = 16
NW = NC * NS

B = 16384
S = 20
D = 64
VT = 2001
VTITLE = 1001
RPW = B // NW
CB = 32
NCHUNK = RPW // CB
PW = D // 2
TPW = S // 2
OTILE = CB * 2 * D
MASK_HI = np.int32(-65536)


def _treemax(vals):
    while len(vals) > 1:
        nxt = [jnp.maximum(a, b) for a, b in zip(vals[::2], vals[1::2])]
        if len(vals) % 2:
            nxt.append(vals[-1])
        vals = nxt
    return vals[0]


def _unpack_store(ai, ref, idx2, base):
    plsc.store_scatter(ref, [idx2 + base],
                       plsc.bitcast(jnp.left_shift(ai, 16), jnp.float32))
    plsc.store_scatter(ref, [idx2 + (base + 1)],
                       plsc.bitcast(jnp.bitwise_and(ai, MASK_HI), jnp.float32))


def _sc_body(title_ids_hbm, tok_hbm, title_tab_hbm, text_tab_hbm,
             out_hbm, textv, titlev, rawv, pkv, tidv, outv, sem, sem_tab):
    wid = lax.axis_index("s") * NC + lax.axis_index("c")

    pltpu.sync_copy(tok_hbm.at[pl.ds(wid * (RPW * S), RPW * S)], rawv)
    pltpu.sync_copy(title_ids_hbm.at[pl.ds(wid * RPW, RPW)],
                    tidv.at[pl.ds(0, RPW)])
    text_dma = pltpu.async_copy(text_tab_hbm, textv, sem_tab)
    title_dma = pltpu.async_copy(title_tab_hbm, titlev, sem_tab)

    iota = lax.iota(jnp.int32, 16)
    idxae = jnp.bitwise_and(iota * 2, jnp.int32(15))
    idxao = jnp.bitwise_and(iota * 2 + 1, jnp.int32(15))
    idxbe = jnp.bitwise_and(iota * 2 - 4, jnp.int32(15))
    idxbo = jnp.bitwise_and(iota * 2 - 3, jnp.int32(15))
    zero16 = jnp.bitwise_and(iota, jnp.int32(0))
    lane_lo = iota < 8
    tok_lane = iota < TPW
    _dn = lax.GatherDimensionNumbers(
        offset_dims=(), collapsed_slice_dims=(0,), start_index_map=(0,))

    def _lanegather(v, i):
        return lax.gather(v, i[:, None], _dn, slice_sizes=(1,),
                          mode=lax.GatherScatterMode.PROMISE_IN_BOUNDS)

    def remap_pack(r, _):
        va = rawv[pl.ds(r * S, 16)]
        vb = rawv[pl.ds(r * S + 4, 16)]
        e = jnp.where(lane_lo, _lanegather(va, idxae), _lanegather(vb, idxbe))
        o = jnp.where(lane_lo, _lanegather(va, idxao), _lanegather(vb, idxbo))
        e = jnp.where(e == 0, jnp.int32(VT - 1), e) * PW
        o = jnp.where(o == 0, jnp.int32(VT - 1), o) * PW
        packed = jnp.bitwise_or(e, jnp.left_shift(o, 16))
        tv = tidv[pl.ds(r, 16)]
        tbc = _lanegather(tv, zero16) * PW
        pkv[pl.ds(r * 16, 16)] = jnp.where(tok_lane, packed, tbc)
        return _

    lax.fori_loop(0, RPW, remap_pack, None)

    text_dma.wait()
    title_dma.wait()

    def chunk_body(g, _):
        ob0 = lax.rem(g, 2) * OTILE

        @pl.when(g >= 2)
        def _wait_prev():
            pltpu.make_async_copy(
                outv.at[pl.ds(0, OTILE)], out_hbm.at[pl.ds(0, OTILE)],
                sem).wait()

        idx2 = iota * 2

        @plsc.parallel_loop(0, CB, 1, unroll=2)
        def _row(r):
            pvi = pkv[pl.ds((g * CB + r) * 16, 16)]
            offs = []
            for k in range(TPW):
                w = pvi[k]
                offs.append(jnp.bitwise_and(w, jnp.int32(0xFFFF)))
                offs.append(lax.shift_right_logical(w, 16))
            ta = pvi[TPW]
            ob = ob0 + r * 2 * D
            for h in (0, 1):
                vals = [
                    plsc.bitcast(textv[pl.ds(offs[s] + h * 16, 16)],
                                 jnp.bfloat16)
                    for s in range(S)
                ]
                acc = plsc.bitcast(_treemax(vals), jnp.int32)
                _unpack_store(acc, outv, idx2, ob + D + h * 32)
                _unpack_store(titlev[pl.ds(ta + h * 16, 16)], outv,
                              idx2, ob + h * 32)
        pltpu.async_copy(
            outv.at[pl.ds(ob0, OTILE)],
            out_hbm.at[pl.ds((wid * RPW + g * CB) * 2 * D, OTILE)], sem)
        return _

    lax.fori_loop(0, NCHUNK, chunk_body, None)
    for _ in range(2):
        pltpu.make_async_copy(
            outv.at[pl.ds(0, OTILE)], out_hbm.at[pl.ds(0, OTILE)], sem).wait()


def _packtab(table_f32):
    v = table_f32.shape[0]
    t = table_f32.astype(jnp.bfloat16).reshape(v, PW, 2)
    return lax.bitcast_convert_type(t, jnp.int32).reshape(-1)


@jax.jit
def _run(title_ids, tok_pk, title_perm, text_perm):
    mesh = plsc.VectorSubcoreMesh(core_axis_name="c", subcore_axis_name="s")
    f = functools.partial(
        pl.kernel,
        out_type=jax.ShapeDtypeStruct((B * 2 * D,), jnp.float32),
        mesh=mesh,
        compiler_params=pltpu.CompilerParams(
            needs_layout_passes=False, disable_bounds_checks=True),
        scratch_types=[
            pltpu.VMEM((VT * PW,), jnp.int32),
            pltpu.VMEM((VTITLE * PW,), jnp.int32),
            pltpu.VMEM((RPW * S,), jnp.int32),
            pltpu.VMEM((RPW * 16,), jnp.int32),
            pltpu.VMEM((RPW + 16,), jnp.int32),
            pltpu.VMEM((2 * OTILE,), jnp.float32),
            pltpu.SemaphoreType.DMA,
            pltpu.SemaphoreType.DMA,
        ],
    )(_sc_body)
    return f(title_ids, tok_pk, title_perm, text_perm)


def kernel(title_ids, token_ids, title_table, text_table):
    text_aug = jnp.concatenate(
        [text_table, jnp.full((1, D), -1e9, jnp.float32)], axis=0)
    text_perm = _packtab(text_aug)
    title_perm = _packtab(title_table)
    out = _run(title_ids, token_ids.reshape(-1), title_perm, text_perm)
    return out.reshape(B, 2 * D)


# speedup vs baseline: 2.2469x; 1.0048x over previous
import functools

import jax
import jax.numpy as jnp
import numpy as np
from jax import lax
from jax.experimental import pallas as pl
from jax.experimental.pallas import tpu as pltpu
from jax.experimental.pallas import tpu_sc as plsc

NC = 2
NS = 16
NW = NC * NS

B = 16384
S = 20
D = 64
VT = 2001
VTITLE = 1001
RPW = B // NW
CB = 32
NCHUNK = RPW // CB
PW = D // 2
TPW = S // 2
OTILE = CB * 2 * D
MASK_HI = np.int32(-65536)


def _treemax(vals):
    while len(vals) > 1:
        nxt = [jnp.maximum(a, b) for a, b in zip(vals[::2], vals[1::2])]
        if len(vals) % 2:
            nxt.append(vals[-1])
        vals = nxt
    return vals[0]


def _unpack_store(ai, ref, idx2, base):
    plsc.store_scatter(ref, [idx2 + base],
                       plsc.bitcast(jnp.left_shift(ai, 16), jnp.float32))
    plsc.store_scatter(ref, [idx2 + (base + 1)],
                       plsc.bitcast(jnp.bitwise_and(ai, MASK_HI), jnp.float32))


def _sc_body(title_ids_hbm, tok_hbm, title_tab_hbm, text_tab_hbm,
             out_hbm, textv, titlev, rawv, pkv, tidv, outv, sem, sem_tab):
    wid = lax.axis_index("s") * NC + lax.axis_index("c")

    pltpu.sync_copy(tok_hbm.at[pl.ds(wid * (RPW * S), RPW * S)], rawv)
    pltpu.sync_copy(title_ids_hbm.at[pl.ds(wid * RPW, RPW)],
                    tidv.at[pl.ds(0, RPW)])
    text_dma = pltpu.async_copy(text_tab_hbm,
                                textv.at[pl.ds(0, (VT - 1) * PW)], sem_tab)
    title_dma = pltpu.async_copy(title_tab_hbm, titlev, sem_tab)
    mrow = plsc.bitcast(jnp.full((32,), -1e9, jnp.bfloat16), jnp.int32)
    textv[pl.ds((VT - 1) * PW, 16)] = mrow
    textv[pl.ds((VT - 1) * PW + 16, 16)] = mrow

    iota = lax.iota(jnp.int32, 16)
    idxae = jnp.bitwise_and(iota * 2, jnp.int32(15))
    idxao = jnp.bitwise_and(iota * 2 + 1, jnp.int32(15))
    idxbe = jnp.bitwise_and(iota * 2 - 4, jnp.int32(15))
    idxbo = jnp.bitwise_and(iota * 2 - 3, jnp.int32(15))
    zero16 = jnp.bitwise_and(iota, jnp.int32(0))
    lane_lo = iota < 8
    tok_lane = iota < TPW
    _dn = lax.GatherDimensionNumbers(
        offset_dims=(), collapsed_slice_dims=(0,), start_index_map=(0,))

    def _lanegather(v, i):
        return lax.gather(v, i[:, None], _dn, slice_sizes=(1,),
                          mode=lax.GatherScatterMode.PROMISE_IN_BOUNDS)

    def remap_pack(r, _):
        va = rawv[pl.ds(r * S, 16)]
        vb = rawv[pl.ds(r * S + 4, 16)]
        e = jnp.where(lane_lo, _lanegather(va, idxae), _lanegather(vb, idxbe))
        o = jnp.where(lane_lo, _lanegather(va, idxao), _lanegather(vb, idxbo))
        e = jnp.where(e == 0, jnp.int32(VT - 1), e) * PW
        o = jnp.where(o == 0, jnp.int32(VT - 1), o) * PW
        packed = jnp.bitwise_or(e, jnp.left_shift(o, 16))
        tv = tidv[pl.ds(r, 16)]
        tbc = _lanegather(tv, zero16) * PW
        pkv[pl.ds(r * 16, 16)] = jnp.where(tok_lane, packed, tbc)
        return _

    lax.fori_loop(0, RPW, remap_pack, None)

    text_dma.wait()
    title_dma.wait()

    def chunk_body(g, _):
        ob0 = lax.rem(g, 2) * OTILE

        @pl.when(g >= 2)
        def _wait_prev():
            pltpu.make_async_copy(
                outv.at[pl.ds(0, OTILE)], out_hbm.at[pl.ds(0, OTILE)],
                sem).wait()

        idx2 = iota * 2

        @plsc.parallel_loop(0, CB, 1, unroll=2)
        def _row(r):
            pvi = pkv[pl.ds((g * CB + r) * 16, 16)]
            offs = []
            for k in range(TPW):
                w = pvi[k]
                offs.append(jnp.bitwise_and(w, jnp.int32(0xFFFF)))
                offs.append(lax.shift_right_logical(w, 16))
            ta = pvi[TPW]
            ob = ob0 + r * 2 * D
            for h in (0, 1):
                vals = [
                    plsc.bitcast(textv[pl.ds(offs[s] + h * 16, 16)],
                                 jnp.bfloat16)
                    for s in range(S)
                ]
                acc = plsc.bitcast(_treemax(vals), jnp.int32)
                _unpack_store(acc, outv, idx2, ob + D + h * 32)
                _unpack_store(titlev[pl.ds(ta + h * 16, 16)], outv,
                              idx2, ob + h * 32)
        pltpu.async_copy(
            outv.at[pl.ds(ob0, OTILE)],
            out_hbm.at[pl.ds((wid * RPW + g * CB) * 2 * D, OTILE)], sem)
        return _

    lax.fori_loop(0, NCHUNK, chunk_body, None)
    for _ in range(2):
        pltpu.make_async_copy(
            outv.at[pl.ds(0, OTILE)], out_hbm.at[pl.ds(0, OTILE)], sem).wait()


def _packtab(table_f32):
    v = table_f32.shape[0]
    t = table_f32.astype(jnp.bfloat16).reshape(v, PW, 2)
    return lax.bitcast_convert_type(t, jnp.int32).reshape(-1)


@jax.jit
def _run(title_ids, tok_pk, title_perm, text_perm):
    mesh = plsc.VectorSubcoreMesh(core_axis_name="c", subcore_axis_name="s")
    f = functools.partial(
        pl.kernel,
        out_type=jax.ShapeDtypeStruct((B * 2 * D,), jnp.float32),
        mesh=mesh,
        compiler_params=pltpu.CompilerParams(
            needs_layout_passes=False, disable_bounds_checks=True),
        scratch_types=[
            pltpu.VMEM((VT * PW,), jnp.int32),
            pltpu.VMEM((VTITLE * PW,), jnp.int32),
            pltpu.VMEM((RPW * S,), jnp.int32),
            pltpu.VMEM((RPW * 16,), jnp.int32),
            pltpu.VMEM((RPW + 16,), jnp.int32),
            pltpu.VMEM((2 * OTILE,), jnp.float32),
            pltpu.SemaphoreType.DMA,
            pltpu.SemaphoreType.DMA,
        ],
    )(_sc_body)
    return f(title_ids, tok_pk, title_perm, text_perm)


def kernel(title_ids, token_ids, title_table, text_table):
    text_perm = _packtab(text_table)
    title_perm = _packtab(title_table)
    out = _run(title_ids, token_ids.reshape(-1), title_perm, text_perm)
    return out.reshape(B, 2 * D)
